# baseline (speedup 1.0000x reference)
import functools

import jax
import jax.numpy as jnp
from jax import lax
from jax.experimental import pallas as pl
from jax.experimental.pallas import tpu as pltpu
from jax.experimental.pallas import tpu_sc as plsc

BATCH = 16384
WORD_DIM = 128
EMB = 16

_NC = 2
_NS = 16
_NW = _NC * _NS
_RPT = BATCH // _NW
_CHUNK = 128
_NCHUNK = _RPT // _CHUNK


def _sc_gather_body(url_p, author_p, utop, atop, raw_u, raw_a,
                    utop_v, atop_v, ubuf, abuf, sem, osem):
    wid = lax.axis_index("s") * _NC + lax.axis_index("c")
    base = wid * _RPT
    pltpu.sync_copy(utop.at[pl.ds(base, _RPT)], utop_v)
    pltpu.sync_copy(atop.at[pl.ds(base, _RPT)], atop_v)
    for c in range(_NCHUNK):
        sl = pl.ds(c * _CHUNK, _CHUNK)
        cu = pltpu.async_copy(url_p.at[utop_v.at[sl]], ubuf, sem)
        ca = pltpu.async_copy(author_p.at[atop_v.at[sl]], abuf, sem)
        out_sl = pl.ds(base + c * _CHUNK, _CHUNK)
        cu.wait()
        wu = pltpu.async_copy(ubuf, raw_u.at[out_sl], osem)
        ca.wait()
        wa = pltpu.async_copy(abuf, raw_a.at[out_sl], osem)
        wu.wait()
        wa.wait()


@functools.cache
def _sc_gather():
    return pl.kernel(
        _sc_gather_body,
        mesh=plsc.VectorSubcoreMesh(core_axis_name="c", subcore_axis_name="s"),
        out_type=[
            jax.ShapeDtypeStruct((BATCH, 128), jnp.float32),
            jax.ShapeDtypeStruct((BATCH, 128), jnp.float32),
        ],
        scratch_types=[
            pltpu.VMEM((_RPT,), jnp.int32),
            pltpu.VMEM((_RPT,), jnp.int32),
            pltpu.VMEM((_CHUNK, 128), jnp.float32),
            pltpu.VMEM((_CHUNK, 128), jnp.float32),
            pltpu.SemaphoreType.DMA,
            pltpu.SemaphoreType.DMA,
        ],
    )


_BM = 4096


def _mlp_body(title, raw_u, raw_a, ucb, acb, w1t, w1u, w1a, b1, w2, b2, w3, b3, out):
    lg = lax.broadcasted_iota(jnp.int32, (_BM, 128), 1) >> 4
    um = jnp.where(lg == ucb[:], raw_u[:], 0.0)
    am = jnp.where(lg == acb[:], raw_a[:], 0.0)
    h1 = (jnp.dot(title[:], w1t[:], preferred_element_type=jnp.float32)
          + jnp.dot(um, w1u[:], preferred_element_type=jnp.float32)
          + jnp.dot(am, w1a[:], preferred_element_type=jnp.float32)
          + b1[:])
    h1 = jnp.maximum(h1, 0.0)
    h2 = jnp.maximum(jnp.dot(h1, w2[:], preferred_element_type=jnp.float32) + b2[:], 0.0)
    out[:] = jnp.sum(h2 * w3[:].reshape(1, 64), axis=1, keepdims=True) + b3[:]


def _mlp(title_emb, raw_u, raw_a, ucb, acb, W1t, W1u_rep, W1a_rep, b1, W2, b2, W3, b3):
    grid = (BATCH // _BM,)
    return pl.pallas_call(
        _mlp_body,
        grid=grid,
        in_specs=[
            pl.BlockSpec((_BM, WORD_DIM), lambda i: (i, 0)),
            pl.BlockSpec((_BM, 128), lambda i: (i, 0)),
            pl.BlockSpec((_BM, 128), lambda i: (i, 0)),
            pl.BlockSpec((_BM, 1), lambda i: (i, 0)),
            pl.BlockSpec((_BM, 1), lambda i: (i, 0)),
            pl.BlockSpec((WORD_DIM, 128), lambda i: (0, 0)),
            pl.BlockSpec((128, 128), lambda i: (0, 0)),
            pl.BlockSpec((128, 128), lambda i: (0, 0)),
            pl.BlockSpec((1, 128), lambda i: (0, 0)),
            pl.BlockSpec((128, 64), lambda i: (0, 0)),
            pl.BlockSpec((1, 64), lambda i: (0, 0)),
            pl.BlockSpec((64, 1), lambda i: (0, 0)),
            pl.BlockSpec((1, 1), lambda i: (0, 0)),
        ],
        out_specs=pl.BlockSpec((_BM, 1), lambda i: (i, 0)),
        out_shape=jax.ShapeDtypeStruct((BATCH, 1), jnp.float32),
    )(title_emb, raw_u, raw_a, ucb, acb, W1t, W1u_rep, W1a_rep, b1, W2, b2, W3, b3)


def kernel(title_emb, url_idx, author_idx, url_table, author_table, W1, b1, W2, b2, W3, b3):
    url_idx = url_idx.astype(jnp.int32)
    author_idx = author_idx.astype(jnp.int32)
    url_p = jnp.pad(url_table, ((0, 7), (0, 0))).reshape(-1, 128)
    author_p = jnp.pad(author_table, ((0, 7), (0, 0))).reshape(-1, 128)
    utop = url_idx >> 3
    atop = author_idx >> 3
    raw_u, raw_a = _sc_gather()(url_p, author_p, utop, atop)
    ucb = (url_idx & 7).reshape(BATCH, 1)
    acb = (author_idx & 7).reshape(BATCH, 1)
    W1t = W1[:WORD_DIM]
    W1u_rep = jnp.tile(W1[WORD_DIM:WORD_DIM + EMB], (8, 1))
    W1a_rep = jnp.tile(W1[WORD_DIM + EMB:], (8, 1))
    return _mlp(title_emb, raw_u, raw_a, ucb, acb, W1t, W1u_rep, W1a_rep,
                b1.reshape(1, 128), W2, b2.reshape(1, 64), W3, b3.reshape(1, 1))


# speedup vs baseline: 1.7138x; 1.7138x over previous
import functools

import jax
import jax.numpy as jnp
from jax import lax
from jax.experimental import pallas as pl
from jax.experimental.pallas import tpu as pltpu
from jax.experimental.pallas import tpu_sc as plsc

BATCH = 16384
WORD_DIM = 128
EMB = 16

_NC = 2
_NS = 16
_NW = _NC * _NS
_RPT = BATCH // _NW
_CHUNK = 128
_NCHUNK = _RPT // _CHUNK


def _sc_gather_body(url_table, author_table, uidx, aidx, url_flat, author_flat,
                    uidx_v, aidx_v, ubuf, abuf, sem, osem):
    wid = lax.axis_index("s") * _NC + lax.axis_index("c")
    base = wid * _RPT
    pltpu.sync_copy(uidx.at[pl.ds(base, _RPT)], uidx_v)
    pltpu.sync_copy(aidx.at[pl.ds(base, _RPT)], aidx_v)
    handles = []
    for c in range(_NCHUNK):
        sl = pl.ds(c * _CHUNK, _CHUNK)
        handles.append(pltpu.async_copy(url_table.at[uidx_v.at[sl]], ubuf.at[sl], sem))
        handles.append(pltpu.async_copy(author_table.at[aidx_v.at[sl]], abuf.at[sl], sem))
    for h in handles:
        h.wait()

    def row_out(i, carry):
        fl = pl.ds((base + i) * 128 + (i % 8) * EMB, EMB)
        pltpu.async_copy(ubuf.at[i], url_flat.at[fl], osem)
        pltpu.async_copy(abuf.at[i], author_flat.at[fl], osem)
        return carry

    lax.fori_loop(0, _RPT, row_out, 0)
    pltpu.make_async_copy(url_table.at[pl.ds(0, _RPT)], ubuf, osem).wait()
    pltpu.make_async_copy(author_table.at[pl.ds(0, _RPT)], abuf, osem).wait()


@functools.cache
def _sc_gather():
    return pl.kernel(
        _sc_gather_body,
        mesh=plsc.VectorSubcoreMesh(core_axis_name="c", subcore_axis_name="s"),
        out_type=[
            jax.ShapeDtypeStruct((BATCH * 128,), jnp.float32),
            jax.ShapeDtypeStruct((BATCH * 128,), jnp.float32),
        ],
        scratch_types=[
            pltpu.VMEM((_RPT,), jnp.int32),
            pltpu.VMEM((_RPT,), jnp.int32),
            pltpu.VMEM((_RPT, EMB), jnp.float32),
            pltpu.VMEM((_RPT, EMB), jnp.float32),
            pltpu.SemaphoreType.DMA,
            pltpu.SemaphoreType.DMA,
        ],
        compiler_params=pltpu.CompilerParams(use_tc_tiling_on_sc=False),
    )


_BM = 4096


def _mlp_body(title, raw_u, raw_a, w1t, w1u, w1a, b1, w2, b2, w3, b3, out):
    lg = lax.broadcasted_iota(jnp.int32, (_BM, 128), 1) >> 4
    rg = lax.broadcasted_iota(jnp.int32, (_BM, 128), 0) & 7
    um = jnp.where(lg == rg, raw_u[:], 0.0)
    am = jnp.where(lg == rg, raw_a[:], 0.0)
    h1 = (jnp.dot(title[:], w1t[:], preferred_element_type=jnp.float32)
          + jnp.dot(um, w1u[:], preferred_element_type=jnp.float32)
          + jnp.dot(am, w1a[:], preferred_element_type=jnp.float32)
          + b1[:])
    h1 = jnp.maximum(h1, 0.0)
    h2 = jnp.maximum(jnp.dot(h1, w2[:], preferred_element_type=jnp.float32) + b2[:], 0.0)
    out[:] = jnp.sum(h2 * w3[:].reshape(1, 64), axis=1, keepdims=True) + b3[:]


def _mlp(title_emb, url_flat, author_flat, W1t, W1u, W1a, b1, W2, b2, W3, b3):
    grid = (BATCH // _BM,)
    return pl.pallas_call(
        _mlp_body,
        grid=grid,
        in_specs=[
            pl.BlockSpec((_BM, WORD_DIM), lambda i: (i, 0)),
            pl.BlockSpec((_BM, 128), lambda i: (i, 0)),
            pl.BlockSpec((_BM, 128), lambda i: (i, 0)),
            pl.BlockSpec((WORD_DIM, 128), lambda i: (0, 0)),
            pl.BlockSpec((128, 128), lambda i: (0, 0)),
            pl.BlockSpec((128, 128), lambda i: (0, 0)),
            pl.BlockSpec((1, 128), lambda i: (0, 0)),
            pl.BlockSpec((128, 64), lambda i: (0, 0)),
            pl.BlockSpec((1, 64), lambda i: (0, 0)),
            pl.BlockSpec((64, 1), lambda i: (0, 0)),
            pl.BlockSpec((1, 1), lambda i: (0, 0)),
        ],
        out_specs=pl.BlockSpec((_BM, 1), lambda i: (i, 0)),
        out_shape=jax.ShapeDtypeStruct((BATCH, 1), jnp.float32),
    )(title_emb, url_flat, author_flat, W1t, W1u, W1a, b1, W2, b2, W3, b3)


def kernel(title_emb, url_idx, author_idx, url_table, author_table, W1, b1, W2, b2, W3, b3):
    url_idx = url_idx.astype(jnp.int32)
    author_idx = author_idx.astype(jnp.int32)
    url_flat, author_flat = _sc_gather()(url_table, author_table, url_idx, author_idx)
    raw_u = url_flat.reshape(BATCH, 128)
    raw_a = author_flat.reshape(BATCH, 128)
    W1t = W1[:WORD_DIM]
    W1u = jnp.tile(W1[WORD_DIM:WORD_DIM + EMB], (8, 1))
    W1a = jnp.tile(W1[WORD_DIM + EMB:], (8, 1))
    return _mlp(title_emb, raw_u, raw_a, W1t, W1u, W1a,
                b1.reshape(1, 128), W2, b2.reshape(1, 64), W3, b3.reshape(1, 1))


# speedup vs baseline: 1.7176x; 1.0023x over previous
import functools

import jax
import jax.numpy as jnp
from jax import lax
from jax.experimental import pallas as pl
from jax.experimental.pallas import tpu as pltpu
from jax.experimental.pallas import tpu_sc as plsc

BATCH = 16384
WORD_DIM = 128
EMB = 16

_NC = 2
_NS = 16
_NW = _NC * _NS
_RPT = BATCH // _NW
_CHUNK = 128
_NCHUNK = _RPT // _CHUNK


def _sc_gather_body(url_table, author_table, uidx, aidx, url_flat, author_flat,
                    uidx_v, aidx_v, ubuf, abuf, sem, osem):
    wid = lax.axis_index("s") * _NC + lax.axis_index("c")
    base = wid * _RPT
    pltpu.sync_copy(uidx.at[pl.ds(base, _RPT)], uidx_v)
    pltpu.sync_copy(aidx.at[pl.ds(base, _RPT)], aidx_v)
    handles = []
    for c in range(_NCHUNK):
        sl = pl.ds(c * _CHUNK, _CHUNK)
        handles.append(pltpu.async_copy(url_table.at[uidx_v.at[sl]], ubuf.at[sl], sem))
        handles.append(pltpu.async_copy(author_table.at[aidx_v.at[sl]], abuf.at[sl], sem))
    for h in handles:
        h.wait()

    def row_out(i, carry):
        fl = pl.ds((i % 8) * EMB, EMB)
        pltpu.async_copy(ubuf.at[i], url_flat.at[base + i, fl], osem)
        pltpu.async_copy(abuf.at[i], author_flat.at[base + i, fl], osem)
        return carry

    lax.fori_loop(0, _RPT, row_out, 0)
    pltpu.make_async_copy(url_table.at[pl.ds(0, _RPT)], ubuf, osem).wait()
    pltpu.make_async_copy(author_table.at[pl.ds(0, _RPT)], abuf, osem).wait()


@functools.cache
def _sc_gather():
    return pl.kernel(
        _sc_gather_body,
        mesh=plsc.VectorSubcoreMesh(core_axis_name="c", subcore_axis_name="s"),
        out_type=[
            jax.ShapeDtypeStruct((BATCH, 128), jnp.float32),
            jax.ShapeDtypeStruct((BATCH, 128), jnp.float32),
        ],
        scratch_types=[
            pltpu.VMEM((_RPT,), jnp.int32),
            pltpu.VMEM((_RPT,), jnp.int32),
            pltpu.VMEM((_RPT, EMB), jnp.float32),
            pltpu.VMEM((_RPT, EMB), jnp.float32),
            pltpu.SemaphoreType.DMA,
            pltpu.SemaphoreType.DMA,
        ],
        compiler_params=pltpu.CompilerParams(use_tc_tiling_on_sc=False),
    )


_BM = 4096


def _mlp_body(title, raw_u, raw_a, w1t, w1u, w1a, b1, w2, b2, w3, b3, out):
    lg = lax.broadcasted_iota(jnp.int32, (_BM, 128), 1) >> 4
    rg = lax.broadcasted_iota(jnp.int32, (_BM, 128), 0) & 7
    um = jnp.where(lg == rg, raw_u[:], 0.0)
    am = jnp.where(lg == rg, raw_a[:], 0.0)
    h1 = (jnp.dot(title[:], w1t[:], preferred_element_type=jnp.float32)
          + jnp.dot(um, w1u[:], preferred_element_type=jnp.float32)
          + jnp.dot(am, w1a[:], preferred_element_type=jnp.float32)
          + b1[:])
    h1 = jnp.maximum(h1, 0.0)
    h2 = jnp.maximum(jnp.dot(h1, w2[:], preferred_element_type=jnp.float32) + b2[:], 0.0)
    out[:] = jnp.sum(h2 * w3[:].reshape(1, 64), axis=1, keepdims=True) + b3[:]


def _mlp(title_emb, url_flat, author_flat, W1t, W1u, W1a, b1, W2, b2, W3, b3):
    grid = (BATCH // _BM,)
    return pl.pallas_call(
        _mlp_body,
        grid=grid,
        in_specs=[
            pl.BlockSpec((_BM, WORD_DIM), lambda i: (i, 0)),
            pl.BlockSpec((_BM, 128), lambda i: (i, 0)),
            pl.BlockSpec((_BM, 128), lambda i: (i, 0)),
            pl.BlockSpec((WORD_DIM, 128), lambda i: (0, 0)),
            pl.BlockSpec((128, 128), lambda i: (0, 0)),
            pl.BlockSpec((128, 128), lambda i: (0, 0)),
            pl.BlockSpec((1, 128), lambda i: (0, 0)),
            pl.BlockSpec((128, 64), lambda i: (0, 0)),
            pl.BlockSpec((1, 64), lambda i: (0, 0)),
            pl.BlockSpec((64, 1), lambda i: (0, 0)),
            pl.BlockSpec((1, 1), lambda i: (0, 0)),
        ],
        out_specs=pl.BlockSpec((_BM, 1), lambda i: (i, 0)),
        out_shape=jax.ShapeDtypeStruct((BATCH, 1), jnp.float32),
    )(title_emb, url_flat, author_flat, W1t, W1u, W1a, b1, W2, b2, W3, b3)


def kernel(title_emb, url_idx, author_idx, url_table, author_table, W1, b1, W2, b2, W3, b3):
    url_idx = url_idx.astype(jnp.int32)
    author_idx = author_idx.astype(jnp.int32)
    raw_u, raw_a = _sc_gather()(url_table, author_table, url_idx, author_idx)
    W1t = W1[:WORD_DIM]
    W1u = jnp.tile(W1[WORD_DIM:WORD_DIM + EMB], (8, 1))
    W1a = jnp.tile(W1[WORD_DIM + EMB:], (8, 1))
    return _mlp(title_emb, raw_u, raw_a, W1t, W1u, W1a,
                b1.reshape(1, 128), W2, b2.reshape(1, 64), W3, b3.reshape(1, 1))
